# baseline (speedup 1.0000x reference)
import functools
import math

import jax
import jax.numpy as jnp
from jax import lax
from jax.experimental import pallas as pl
from jax.experimental.pallas import tpu as pltpu
from jax.experimental.pallas import tpu_sc as plsc

RCUT = 5.0
NC = 2
NS = 16
L = 16


def _edge_scalar_body(r_ref, ux_ref, uy_ref, uz_ref,
                      er_ref, c_ref, s0_ref, s1_ref, s2_ref, s3_ref, s4_ref):
    r = r_ref[...]
    alpha = 5.0 / RCUT
    er_ref[...] = jnp.exp(-alpha * r)
    c = 0.5 * (jnp.cos(math.pi / RCUT * r) + 1.0)
    c_ref[...] = c * (r < RCUT).astype(jnp.float32)
    x = ux_ref[...]
    y = uy_ref[...]
    zc = uz_ref[...]
    s3 = math.sqrt(3.0)
    s0_ref[...] = s3 * x * y
    s1_ref[...] = s3 * y * zc
    s2_ref[...] = 0.5 * (3.0 * zc * zc - 1.0)
    s3_ref[...] = s3 * zc * x
    s4_ref[...] = 0.5 * s3 * (x * x - y * y)


def _edge_dense_body(er_ref, c_ref, wdz_ref, wez_ref, mn_ref, bt_ref,
                     gate_ref, erp_ref):
    ps = er_ref.shape[1]
    z5 = jnp.zeros((5, 128), jnp.float32)
    one = jnp.ones((1, 128), jnp.float32)
    mn = mn_ref[...]
    bt = bt_ref[...]
    dn = (((0,), (0,)), ((), ()))
    for g in range(ps):
        er_r = er_ref[0, g][None, :]
        c_r = c_ref[0, g][None, :]
        d = er_r - mn
        rbf = jnp.exp(-bt * d * d)
        ag = jnp.concatenate([rbf * c_r, c_r, z5], axis=0)
        a2 = jnp.concatenate([rbf, one, z5], axis=0)
        gate_ref[pl.ds(g * 128, 128), :] = lax.dot_general(
            ag, wdz_ref[...], dn, precision=lax.Precision.HIGHEST,
            preferred_element_type=jnp.float32)
        erp_ref[pl.ds(g * 128, 128), :] = lax.dot_general(
            a2, wez_ref[...], dn, precision=lax.Precision.HIGHEST,
            preferred_element_type=jnp.float32)


def _node_feat_body(z_ref, wnbr_ref, bnbr_ref, wna_ref, bna_ref,
                    nbr_ref, na_ref):
    z = z_ref[...]
    nbr_ref[...] = jax.nn.silu(
        jnp.dot(z, wnbr_ref[...], preferred_element_type=jnp.float32)
        + bnbr_ref[...])
    na_ref[...] = jax.nn.silu(
        jnp.dot(z, wna_ref[...], preferred_element_type=jnp.float32)
        + bna_ref[...])


def _h_body(na_ref, m2_ref, wa_ref, wm_ref, b_ref, g_ref, lb_ref,
            wru_ref, bru_ref, h_ref):
    m = m2_ref[0] + m2_ref[1]
    h = (jnp.dot(na_ref[...], wa_ref[...], preferred_element_type=jnp.float32)
         + jnp.dot(m, wm_ref[...], preferred_element_type=jnp.float32)
         + b_ref[...])
    mu = jnp.mean(h, axis=-1, keepdims=True)
    var = jnp.mean((h - mu) ** 2, axis=-1, keepdims=True)
    h = (h - mu) * lax.rsqrt(var + 1e-5) * g_ref[...] + lb_ref[...]
    h = jax.nn.silu(h)
    h_ref[...] = (
        jnp.dot(h, wru_ref[...], preferred_element_type=jnp.float32)
        + bru_ref[...])


def _sc_msg_body(nbr_hbm, gate_hbm, isrc_hbm, jsrc_hbm, zeros_hbm, out_hbm,
                 jall, iv0, iv1, r0, r1, g0, g1, s0, s1, msh,
                 sg0, sg1, siv0, siv1, ssc0, ssc1, *, n_pad, dne, epw, k, nch):
    cid = lax.axis_index("c")
    sid = lax.axis_index("s")
    wid = sid * NC + cid
    rpt = n_pad // NS
    base = wid * epw
    bufs = ((iv0, r0, g0, s0, sg0, siv0, ssc0),
            (iv1, r1, g1, s1, sg1, siv1, ssc1))

    def fire(b, g):
        iv, rows, gv, sv, sg, siv, _ = bufs[b]
        off = base + g * k
        pltpu.async_copy(nbr_hbm.at[jall.at[pl.ds(g * k, k)]], rows, sg)
        pltpu.async_copy(gate_hbm.at[pl.ds(off, k)], gv, sg)

    def process(b, g, first):
        iv, rows, gv, sv, sg, siv, ssc = bufs[b]
        off = base + g * k
        pltpu.make_async_copy(
            nbr_hbm.at[jall.at[pl.ds(g * k, k)]], rows, sg).wait()
        pltpu.make_async_copy(gate_hbm.at[pl.ds(off, k)], gv, sg).wait()
        if not first:
            pltpu.make_async_copy(sv, msh.at[iv], ssc).wait()
        pltpu.async_copy(isrc_hbm.at[pl.ds(off, k)], iv, siv)

        @plsc.parallel_loop(0, k, unroll=4)
        def rowfn(rr):
            for cc in range(dne // L):
                sl = pl.ds(cc * L, L)
                sv[rr, sl] = rows[rr, sl] * gv[rr, sl]
        pltpu.make_async_copy(isrc_hbm.at[pl.ds(off, k)], iv, siv).wait()
        pltpu.async_copy(sv, msh.at[iv], ssc, add=True)

    def waitsc(b):
        iv, rows, gv, sv, sg, siv, ssc = bufs[b]
        pltpu.make_async_copy(sv, msh.at[iv], ssc).wait()

    pltpu.sync_copy(jsrc_hbm.at[pl.ds(base, epw)], jall)
    pltpu.sync_copy(zeros_hbm.at[pl.ds(sid * rpt, rpt)],
                    msh.at[pl.ds(sid * rpt, rpt)])
    fire(0, 0)
    fire(1, 1)
    plsc.subcore_barrier()
    process(0, 0, True)
    fire(0, 2)
    process(1, 1, True)
    fire(1, 3)

    def pair(p, carry):
        process(0, 2 * p, False)
        fire(0, 2 * p + 2)
        process(1, 2 * p + 1, False)
        fire(1, 2 * p + 3)
        return carry

    lax.fori_loop(1, (nch - 2) // 2, pair, 0)
    process(0, nch - 2, False)
    process(1, nch - 1, False)
    waitsc(0)
    waitsc(1)
    plsc.subcore_barrier()
    pltpu.sync_copy(msh.at[pl.ds(sid * rpt, rpt)],
                    out_hbm.at[cid, pl.ds(sid * rpt, rpt)])


def _sc_t_body(h_hbm, erp_hbm, isrc_hbm, jsrc_hbm, t_hbm,
               iall, jall, a0, a1, b0, b1, e0, e1, s0, s1,
               sg0, sg1, st0, st1, *, dne, epw, k, nch):
    cid = lax.axis_index("c")
    sid = lax.axis_index("s")
    wid = sid * NC + cid
    base = wid * epw
    bufs = ((a0, b0, e0, s0, sg0, st0), (a1, b1, e1, s1, sg1, st1))

    def fire(b, g):
        av, bv, ev, sv, sg, _ = bufs[b]
        off = base + g * k
        pltpu.async_copy(h_hbm.at[iall.at[pl.ds(g * k, k)]], av, sg)
        pltpu.async_copy(h_hbm.at[jall.at[pl.ds(g * k, k)]], bv, sg)
        pltpu.async_copy(erp_hbm.at[pl.ds(off, k)], ev, sg)

    def process(b, g, first):
        av, bv, ev, sv, sg, st = bufs[b]
        off = base + g * k
        pltpu.make_async_copy(
            h_hbm.at[iall.at[pl.ds(g * k, k)]], av, sg).wait()
        pltpu.make_async_copy(
            h_hbm.at[jall.at[pl.ds(g * k, k)]], bv, sg).wait()
        pltpu.make_async_copy(erp_hbm.at[pl.ds(off, k)], ev, sg).wait()
        if not first:
            pltpu.make_async_copy(sv, t_hbm.at[pl.ds(off, k)], st).wait()

        @plsc.parallel_loop(0, k, unroll=4)
        def rowfn(rr):
            for cc in range(dne // L):
                sl = pl.ds(cc * L, L)
                sv[rr, sl] = (av[rr, sl] + bv[rr, sl]) * ev[rr, sl]
        pltpu.async_copy(sv, t_hbm.at[pl.ds(off, k)], st)

    def waitst(b):
        av, bv, ev, sv, sg, st = bufs[b]
        pltpu.make_async_copy(sv, t_hbm.at[pl.ds(base, k)], st).wait()

    pltpu.sync_copy(isrc_hbm.at[pl.ds(base, epw)], iall)
    pltpu.sync_copy(jsrc_hbm.at[pl.ds(base, epw)], jall)
    fire(0, 0)
    fire(1, 1)
    process(0, 0, True)
    fire(0, 2)
    process(1, 1, True)
    fire(1, 3)

    def pair(p, carry):
        process(0, 2 * p, False)
        fire(0, 2 * p + 2)
        process(1, 2 * p + 1, False)
        fire(1, 2 * p + 3)
        return carry

    lax.fori_loop(1, (nch - 3) // 2, pair, 0)
    process(0, nch - 3, False)
    fire(0, nch - 1)
    process(1, nch - 2, False)
    process(0, nch - 1, False)
    waitst(1)
    waitst(0)


def kernel(z, edge, r, u, A_na_w, A_na_b, A_nbr_w, A_nbr_b, W_ndp_w, W_ndp_b,
           W_nrd_w, W_nrd_b, W_nru_w, W_nru_b, W_erp_w, W_erp_b, ln_g, ln_b,
           means, betas):
    n_nodes, zf = z.shape
    n_edges = r.shape[0]
    dne = A_na_w.shape[0]
    nrbf = means.shape[0]

    i_idx = edge[:, 0]
    j_idx = edge[:, 1]
    r2 = r[:, None]

    full = lambda s: pl.BlockSpec(s, lambda i: (0,) * len(s))

    BE = 2560
    n_eb = n_edges // BE
    PS = BE // 128
    pk = lambda a: a.reshape(n_eb, PS, 128)
    pspec = full((n_eb, PS, 128))
    er_p, c_p, s0, s1, s2, s3c, s4 = pl.pallas_call(
        _edge_scalar_body,
        grid=(1,),
        in_specs=[pspec] * 4,
        out_specs=[pspec] * 7,
        out_shape=[jax.ShapeDtypeStruct((n_eb, PS, 128), jnp.float32)] * 7,
    )(pk(r), pk(u[:, 0]), pk(u[:, 1]), pk(u[:, 2]))
    c = c_p.reshape(n_edges, 1)
    rt2 = jnp.stack([s0.reshape(-1), s1.reshape(-1), s2.reshape(-1),
                     s3c.reshape(-1), s4.reshape(-1)], axis=-1)

    gate, erp = pl.pallas_call(
        _edge_dense_body,
        grid=(n_eb,),
        in_specs=[
            pl.BlockSpec((1, PS, 128), lambda i: (i, 0, 0)),
            pl.BlockSpec((1, PS, 128), lambda i: (i, 0, 0)),
            full((nrbf + 6, dne)), full((nrbf + 6, dne)),
            full((nrbf, 1)), full((nrbf, 1)),
        ],
        out_specs=[
            pl.BlockSpec((BE, dne), lambda i: (i, 0)),
            pl.BlockSpec((BE, dne), lambda i: (i, 0)),
        ],
        out_shape=[
            jax.ShapeDtypeStruct((n_edges, dne), jnp.float32),
            jax.ShapeDtypeStruct((n_edges, dne), jnp.float32),
        ],
    )(er_p, c_p,
      jnp.concatenate([W_ndp_w.T, W_ndp_b[None, :],
                       jnp.zeros((5, dne), jnp.float32)], axis=0),
      jnp.concatenate([W_erp_w.T, W_erp_b[None, :],
                       jnp.zeros((5, dne), jnp.float32)], axis=0),
      means[:, None], betas[:, None])

    BN = 2000
    n_nb = n_nodes // BN
    nbr, na = pl.pallas_call(
        _node_feat_body,
        grid=(n_nb,),
        in_specs=[
            pl.BlockSpec((BN, zf), lambda i: (i, 0)),
            full((zf, dne)), full((1, dne)),
            full((zf, dne)), full((1, dne)),
        ],
        out_specs=[
            pl.BlockSpec((BN, dne), lambda i: (i, 0)),
            pl.BlockSpec((BN, dne), lambda i: (i, 0)),
        ],
        out_shape=[
            jax.ShapeDtypeStruct((n_nodes, dne), jnp.float32),
            jax.ShapeDtypeStruct((n_nodes, dne), jnp.float32),
        ],
    )(z, A_nbr_w.T, A_nbr_b[None, :], A_na_w.T, A_na_b[None, :])

    EPW = n_edges // (NC * NS)
    K1 = 40
    NCH1 = EPW // K1
    K2 = 80
    NCH2 = EPW // K2
    n_pad = ((n_nodes + 8 * NS - 1) // (8 * NS)) * (8 * NS)
    mesh = plsc.VectorSubcoreMesh(core_axis_name="c", subcore_axis_name="s",
                                  num_cores=NC, num_subcores=NS)
    zeros = jnp.zeros((n_pad, dne), jnp.float32)
    m2 = pl.kernel(
        functools.partial(_sc_msg_body, n_pad=n_pad, dne=dne, epw=EPW,
                          k=K1, nch=NCH1),
        out_type=jax.ShapeDtypeStruct((NC, n_pad, dne), jnp.float32),
        mesh=mesh,
        scratch_types=[
            pltpu.VMEM((EPW,), jnp.int32),
            pltpu.VMEM((K1,), jnp.int32),
            pltpu.VMEM((K1,), jnp.int32),
            pltpu.VMEM((K1, dne), jnp.float32),
            pltpu.VMEM((K1, dne), jnp.float32),
            pltpu.VMEM((K1, dne), jnp.float32),
            pltpu.VMEM((K1, dne), jnp.float32),
            pltpu.VMEM((K1, dne), jnp.float32),
            pltpu.VMEM((K1, dne), jnp.float32),
            pltpu.VMEM_SHARED((n_pad, dne), jnp.float32),
            pltpu.SemaphoreType.DMA,
            pltpu.SemaphoreType.DMA,
            pltpu.SemaphoreType.DMA,
            pltpu.SemaphoreType.DMA,
            pltpu.SemaphoreType.DMA,
            pltpu.SemaphoreType.DMA,
        ],
    )(nbr, gate, i_idx, j_idx, zeros)

    wa = W_nrd_w[:, :dne].T
    wm = W_nrd_w[:, dne:].T
    (h,) = pl.pallas_call(
        _h_body,
        grid=(n_nb,),
        in_specs=[
            pl.BlockSpec((BN, dne), lambda i: (i, 0)),
            pl.BlockSpec((NC, BN, dne), lambda i: (0, i, 0)),
            full((dne, dne)), full((dne, dne)), full((1, dne)),
            full((1, dne)), full((1, dne)),
            full((dne, dne)), full((1, dne)),
        ],
        out_specs=[pl.BlockSpec((BN, dne), lambda i: (i, 0))],
        out_shape=[jax.ShapeDtypeStruct((n_nodes, dne), jnp.float32)],
    )(na, m2, wa, wm, W_nrd_b[None, :], ln_g[None, :], ln_b[None, :],
      W_nru_w.T, W_nru_b[None, :])

    t = pl.kernel(
        functools.partial(_sc_t_body, dne=dne, epw=EPW, k=K2, nch=NCH2),
        out_type=jax.ShapeDtypeStruct((n_edges, dne), jnp.float32),
        mesh=mesh,
        scratch_types=[
            pltpu.VMEM((EPW,), jnp.int32),
            pltpu.VMEM((EPW,), jnp.int32),
            pltpu.VMEM((K2, dne), jnp.float32),
            pltpu.VMEM((K2, dne), jnp.float32),
            pltpu.VMEM((K2, dne), jnp.float32),
            pltpu.VMEM((K2, dne), jnp.float32),
            pltpu.VMEM((K2, dne), jnp.float32),
            pltpu.VMEM((K2, dne), jnp.float32),
            pltpu.VMEM((K2, dne), jnp.float32),
            pltpu.VMEM((K2, dne), jnp.float32),
            pltpu.SemaphoreType.DMA,
            pltpu.SemaphoreType.DMA,
            pltpu.SemaphoreType.DMA,
            pltpu.SemaphoreType.DMA,
        ],
    )(h, erp, i_idx, j_idx)

    return h, t, r2, u, rt2, c


# speedup vs baseline: 1.2102x; 1.2102x over previous
import functools
import math

import jax
import jax.numpy as jnp
from jax import lax
from jax.experimental import pallas as pl
from jax.experimental.pallas import tpu as pltpu
from jax.experimental.pallas import tpu_sc as plsc

RCUT = 5.0
NC = 2
NS = 16
L = 16


def _edge_scalar_body(r_ref, ux_ref, uy_ref, uz_ref,
                      er_ref, c_ref, s0_ref, s1_ref, s2_ref, s3_ref, s4_ref):
    r = r_ref[...]
    alpha = 5.0 / RCUT
    er_ref[...] = jnp.exp(-alpha * r)
    c = 0.5 * (jnp.cos(math.pi / RCUT * r) + 1.0)
    c_ref[...] = c * (r < RCUT).astype(jnp.float32)
    x = ux_ref[...]
    y = uy_ref[...]
    zc = uz_ref[...]
    s3 = math.sqrt(3.0)
    s0_ref[...] = s3 * x * y
    s1_ref[...] = s3 * y * zc
    s2_ref[...] = 0.5 * (3.0 * zc * zc - 1.0)
    s3_ref[...] = s3 * zc * x
    s4_ref[...] = 0.5 * s3 * (x * x - y * y)


def _edge_dense_body(er_ref, c_ref, wdz_ref, wez_ref, mn_ref, bt_ref,
                     gate_ref, erp_ref):
    ps = er_ref.shape[1]
    z5 = jnp.zeros((5, 128), jnp.float32)
    one = jnp.ones((1, 128), jnp.float32)
    mn = mn_ref[...]
    bt = bt_ref[...]
    dn = (((0,), (0,)), ((), ()))
    for g in range(ps):
        er_r = er_ref[0, g][None, :]
        c_r = c_ref[0, g][None, :]
        d = er_r - mn
        rbf = jnp.exp(-bt * d * d)
        ag = jnp.concatenate([rbf * c_r, c_r, z5], axis=0)
        a2 = jnp.concatenate([rbf, one, z5], axis=0)
        gate_ref[pl.ds(g * 128, 128), :] = lax.dot_general(
            ag, wdz_ref[...], dn, preferred_element_type=jnp.float32)
        erp_ref[pl.ds(g * 128, 128), :] = lax.dot_general(
            a2, wez_ref[...], dn, preferred_element_type=jnp.float32)


def _node_feat_body(z_ref, wnbr_ref, bnbr_ref, wna_ref, bna_ref,
                    nbr_ref, na_ref):
    z = z_ref[...]
    nbr_ref[...] = jax.nn.silu(
        jnp.dot(z, wnbr_ref[...], preferred_element_type=jnp.float32)
        + bnbr_ref[...])
    na_ref[...] = jax.nn.silu(
        jnp.dot(z, wna_ref[...], preferred_element_type=jnp.float32)
        + bna_ref[...])


def _h_body(na_ref, m2_ref, wa_ref, wm_ref, b_ref, g_ref, lb_ref,
            wru_ref, bru_ref, h_ref):
    m = m2_ref[0] + m2_ref[1]
    h = (jnp.dot(na_ref[...], wa_ref[...], preferred_element_type=jnp.float32)
         + jnp.dot(m, wm_ref[...], preferred_element_type=jnp.float32)
         + b_ref[...])
    mu = jnp.mean(h, axis=-1, keepdims=True)
    var = jnp.mean((h - mu) ** 2, axis=-1, keepdims=True)
    h = (h - mu) * lax.rsqrt(var + 1e-5) * g_ref[...] + lb_ref[...]
    h = jax.nn.silu(h)
    h_ref[...] = (
        jnp.dot(h, wru_ref[...], preferred_element_type=jnp.float32)
        + bru_ref[...])


def _sc_msg_body(nbr_hbm, gate_hbm, isrc_hbm, jsrc_hbm, zeros_hbm, out_hbm,
                 jall, iv0, iv1, r0, r1, g0, g1, s0, s1, msh,
                 sg0, sg1, siv0, siv1, ssc0, ssc1, *, n_pad, dne, epw, k, nch):
    cid = lax.axis_index("c")
    sid = lax.axis_index("s")
    wid = sid * NC + cid
    rpt = n_pad // NS
    base = wid * epw
    bufs = ((iv0, r0, g0, s0, sg0, siv0, ssc0),
            (iv1, r1, g1, s1, sg1, siv1, ssc1))

    def fire(b, g):
        iv, rows, gv, sv, sg, siv, _ = bufs[b]
        off = base + g * k
        pltpu.async_copy(nbr_hbm.at[jall.at[pl.ds(g * k, k)]], rows, sg)
        pltpu.async_copy(gate_hbm.at[pl.ds(off, k)], gv, sg)

    def process(b, g, first):
        iv, rows, gv, sv, sg, siv, ssc = bufs[b]
        off = base + g * k
        pltpu.make_async_copy(
            nbr_hbm.at[jall.at[pl.ds(g * k, k)]], rows, sg).wait()
        pltpu.make_async_copy(gate_hbm.at[pl.ds(off, k)], gv, sg).wait()
        if not first:
            pltpu.make_async_copy(sv, msh.at[iv], ssc).wait()
        pltpu.async_copy(isrc_hbm.at[pl.ds(off, k)], iv, siv)

        @plsc.parallel_loop(0, k, unroll=4)
        def rowfn(rr):
            for cc in range(dne // L):
                sl = pl.ds(cc * L, L)
                sv[rr, sl] = rows[rr, sl] * gv[rr, sl]
        pltpu.make_async_copy(isrc_hbm.at[pl.ds(off, k)], iv, siv).wait()
        pltpu.async_copy(sv, msh.at[iv], ssc, add=True)

    def waitsc(b):
        iv, rows, gv, sv, sg, siv, ssc = bufs[b]
        pltpu.make_async_copy(sv, msh.at[iv], ssc).wait()

    pltpu.sync_copy(jsrc_hbm.at[pl.ds(base, epw)], jall)
    pltpu.sync_copy(zeros_hbm.at[pl.ds(sid * rpt, rpt)],
                    msh.at[pl.ds(sid * rpt, rpt)])
    fire(0, 0)
    fire(1, 1)
    plsc.subcore_barrier()
    process(0, 0, True)
    fire(0, 2)
    process(1, 1, True)
    fire(1, 3)

    def pair(p, carry):
        process(0, 2 * p, False)
        fire(0, 2 * p + 2)
        process(1, 2 * p + 1, False)
        fire(1, 2 * p + 3)
        return carry

    lax.fori_loop(1, (nch - 2) // 2, pair, 0)
    process(0, nch - 2, False)
    process(1, nch - 1, False)
    waitsc(0)
    waitsc(1)
    plsc.subcore_barrier()
    pltpu.sync_copy(msh.at[pl.ds(sid * rpt, rpt)],
                    out_hbm.at[cid, pl.ds(sid * rpt, rpt)])


def _sc_t_body(h_hbm, erp_hbm, isrc_hbm, jsrc_hbm, t_hbm,
               iall, jall, a0, a1, b0, b1, e0, e1, s0, s1,
               sg0, sg1, st0, st1, *, dne, epw, k, nch):
    cid = lax.axis_index("c")
    sid = lax.axis_index("s")
    wid = sid * NC + cid
    base = wid * epw
    bufs = ((a0, b0, e0, s0, sg0, st0), (a1, b1, e1, s1, sg1, st1))

    def fire(b, g):
        av, bv, ev, sv, sg, _ = bufs[b]
        off = base + g * k
        pltpu.async_copy(h_hbm.at[iall.at[pl.ds(g * k, k)]], av, sg)
        pltpu.async_copy(h_hbm.at[jall.at[pl.ds(g * k, k)]], bv, sg)
        pltpu.async_copy(erp_hbm.at[pl.ds(off, k)], ev, sg)

    def process(b, g, first):
        av, bv, ev, sv, sg, st = bufs[b]
        off = base + g * k
        pltpu.make_async_copy(
            h_hbm.at[iall.at[pl.ds(g * k, k)]], av, sg).wait()
        pltpu.make_async_copy(
            h_hbm.at[jall.at[pl.ds(g * k, k)]], bv, sg).wait()
        pltpu.make_async_copy(erp_hbm.at[pl.ds(off, k)], ev, sg).wait()
        if not first:
            pltpu.make_async_copy(sv, t_hbm.at[pl.ds(off, k)], st).wait()

        @plsc.parallel_loop(0, k, unroll=4)
        def rowfn(rr):
            for cc in range(dne // L):
                sl = pl.ds(cc * L, L)
                sv[rr, sl] = (av[rr, sl] + bv[rr, sl]) * ev[rr, sl]
        pltpu.async_copy(sv, t_hbm.at[pl.ds(off, k)], st)

    def waitst(b):
        av, bv, ev, sv, sg, st = bufs[b]
        pltpu.make_async_copy(sv, t_hbm.at[pl.ds(base, k)], st).wait()

    pltpu.sync_copy(isrc_hbm.at[pl.ds(base, epw)], iall)
    pltpu.sync_copy(jsrc_hbm.at[pl.ds(base, epw)], jall)
    fire(0, 0)
    fire(1, 1)
    process(0, 0, True)
    fire(0, 2)
    process(1, 1, True)
    fire(1, 3)

    def pair(p, carry):
        process(0, 2 * p, False)
        fire(0, 2 * p + 2)
        process(1, 2 * p + 1, False)
        fire(1, 2 * p + 3)
        return carry

    lax.fori_loop(1, (nch - 3) // 2, pair, 0)
    process(0, nch - 3, False)
    fire(0, nch - 1)
    process(1, nch - 2, False)
    process(0, nch - 1, False)
    waitst(1)
    waitst(0)


def kernel(z, edge, r, u, A_na_w, A_na_b, A_nbr_w, A_nbr_b, W_ndp_w, W_ndp_b,
           W_nrd_w, W_nrd_b, W_nru_w, W_nru_b, W_erp_w, W_erp_b, ln_g, ln_b,
           means, betas):
    n_nodes, zf = z.shape
    n_edges = r.shape[0]
    dne = A_na_w.shape[0]
    nrbf = means.shape[0]

    i_idx = edge[:, 0]
    j_idx = edge[:, 1]
    r2 = r[:, None]

    full = lambda s: pl.BlockSpec(s, lambda i: (0,) * len(s))

    BE = 2560
    n_eb = n_edges // BE
    PS = BE // 128
    pk = lambda a: a.reshape(n_eb, PS, 128)
    pspec = full((n_eb, PS, 128))
    er_p, c_p, s0, s1, s2, s3c, s4 = pl.pallas_call(
        _edge_scalar_body,
        grid=(1,),
        in_specs=[pspec] * 4,
        out_specs=[pspec] * 7,
        out_shape=[jax.ShapeDtypeStruct((n_eb, PS, 128), jnp.float32)] * 7,
    )(pk(r), pk(u[:, 0]), pk(u[:, 1]), pk(u[:, 2]))
    c = c_p.reshape(n_edges, 1)
    rt2 = jnp.stack([s0.reshape(-1), s1.reshape(-1), s2.reshape(-1),
                     s3c.reshape(-1), s4.reshape(-1)], axis=-1)

    gate, erp = pl.pallas_call(
        _edge_dense_body,
        grid=(n_eb,),
        in_specs=[
            pl.BlockSpec((1, PS, 128), lambda i: (i, 0, 0)),
            pl.BlockSpec((1, PS, 128), lambda i: (i, 0, 0)),
            full((nrbf + 6, dne)), full((nrbf + 6, dne)),
            full((nrbf, 1)), full((nrbf, 1)),
        ],
        out_specs=[
            pl.BlockSpec((BE, dne), lambda i: (i, 0)),
            pl.BlockSpec((BE, dne), lambda i: (i, 0)),
        ],
        out_shape=[
            jax.ShapeDtypeStruct((n_edges, dne), jnp.float32),
            jax.ShapeDtypeStruct((n_edges, dne), jnp.float32),
        ],
    )(er_p, c_p,
      jnp.concatenate([W_ndp_w.T, W_ndp_b[None, :],
                       jnp.zeros((5, dne), jnp.float32)], axis=0),
      jnp.concatenate([W_erp_w.T, W_erp_b[None, :],
                       jnp.zeros((5, dne), jnp.float32)], axis=0),
      means[:, None], betas[:, None])

    BN = 2000
    n_nb = n_nodes // BN
    nbr, na = pl.pallas_call(
        _node_feat_body,
        grid=(n_nb,),
        in_specs=[
            pl.BlockSpec((BN, zf), lambda i: (i, 0)),
            full((zf, dne)), full((1, dne)),
            full((zf, dne)), full((1, dne)),
        ],
        out_specs=[
            pl.BlockSpec((BN, dne), lambda i: (i, 0)),
            pl.BlockSpec((BN, dne), lambda i: (i, 0)),
        ],
        out_shape=[
            jax.ShapeDtypeStruct((n_nodes, dne), jnp.float32),
            jax.ShapeDtypeStruct((n_nodes, dne), jnp.float32),
        ],
    )(z, A_nbr_w.T, A_nbr_b[None, :], A_na_w.T, A_na_b[None, :])

    EPW = n_edges // (NC * NS)
    K1 = 40
    NCH1 = EPW // K1
    K2 = 80
    NCH2 = EPW // K2
    n_pad = ((n_nodes + 8 * NS - 1) // (8 * NS)) * (8 * NS)
    mesh = plsc.VectorSubcoreMesh(core_axis_name="c", subcore_axis_name="s",
                                  num_cores=NC, num_subcores=NS)
    zeros = jnp.zeros((n_pad, dne), jnp.float32)
    m2 = pl.kernel(
        functools.partial(_sc_msg_body, n_pad=n_pad, dne=dne, epw=EPW,
                          k=K1, nch=NCH1),
        out_type=jax.ShapeDtypeStruct((NC, n_pad, dne), jnp.float32),
        mesh=mesh,
        scratch_types=[
            pltpu.VMEM((EPW,), jnp.int32),
            pltpu.VMEM((K1,), jnp.int32),
            pltpu.VMEM((K1,), jnp.int32),
            pltpu.VMEM((K1, dne), jnp.float32),
            pltpu.VMEM((K1, dne), jnp.float32),
            pltpu.VMEM((K1, dne), jnp.float32),
            pltpu.VMEM((K1, dne), jnp.float32),
            pltpu.VMEM((K1, dne), jnp.float32),
            pltpu.VMEM((K1, dne), jnp.float32),
            pltpu.VMEM_SHARED((n_pad, dne), jnp.float32),
            pltpu.SemaphoreType.DMA,
            pltpu.SemaphoreType.DMA,
            pltpu.SemaphoreType.DMA,
            pltpu.SemaphoreType.DMA,
            pltpu.SemaphoreType.DMA,
            pltpu.SemaphoreType.DMA,
        ],
    )(nbr, gate, i_idx, j_idx, zeros)

    wa = W_nrd_w[:, :dne].T
    wm = W_nrd_w[:, dne:].T
    (h,) = pl.pallas_call(
        _h_body,
        grid=(n_nb,),
        in_specs=[
            pl.BlockSpec((BN, dne), lambda i: (i, 0)),
            pl.BlockSpec((NC, BN, dne), lambda i: (0, i, 0)),
            full((dne, dne)), full((dne, dne)), full((1, dne)),
            full((1, dne)), full((1, dne)),
            full((dne, dne)), full((1, dne)),
        ],
        out_specs=[pl.BlockSpec((BN, dne), lambda i: (i, 0))],
        out_shape=[jax.ShapeDtypeStruct((n_nodes, dne), jnp.float32)],
    )(na, m2, wa, wm, W_nrd_b[None, :], ln_g[None, :], ln_b[None, :],
      W_nru_w.T, W_nru_b[None, :])

    t = pl.kernel(
        functools.partial(_sc_t_body, dne=dne, epw=EPW, k=K2, nch=NCH2),
        out_type=jax.ShapeDtypeStruct((n_edges, dne), jnp.float32),
        mesh=mesh,
        scratch_types=[
            pltpu.VMEM((EPW,), jnp.int32),
            pltpu.VMEM((EPW,), jnp.int32),
            pltpu.VMEM((K2, dne), jnp.float32),
            pltpu.VMEM((K2, dne), jnp.float32),
            pltpu.VMEM((K2, dne), jnp.float32),
            pltpu.VMEM((K2, dne), jnp.float32),
            pltpu.VMEM((K2, dne), jnp.float32),
            pltpu.VMEM((K2, dne), jnp.float32),
            pltpu.VMEM((K2, dne), jnp.float32),
            pltpu.VMEM((K2, dne), jnp.float32),
            pltpu.SemaphoreType.DMA,
            pltpu.SemaphoreType.DMA,
            pltpu.SemaphoreType.DMA,
            pltpu.SemaphoreType.DMA,
        ],
    )(h, erp, i_idx, j_idx)

    return h, t, r2, u, rt2, c


# speedup vs baseline: 1.2460x; 1.0296x over previous
import functools
import math

import jax
import jax.numpy as jnp
from jax import lax
from jax.experimental import pallas as pl
from jax.experimental.pallas import tpu as pltpu
from jax.experimental.pallas import tpu_sc as plsc

RCUT = 5.0
NC = 2
NS = 16
L = 16


def _edge_scalar_body(r_ref, ux_ref, uy_ref, uz_ref,
                      er_ref, c_ref, s0_ref, s1_ref, s2_ref, s3_ref, s4_ref):
    r = r_ref[...]
    alpha = 5.0 / RCUT
    er_ref[...] = jnp.exp(-alpha * r)
    c = 0.5 * (jnp.cos(math.pi / RCUT * r) + 1.0)
    c_ref[...] = c * (r < RCUT).astype(jnp.float32)
    x = ux_ref[...]
    y = uy_ref[...]
    zc = uz_ref[...]
    s3 = math.sqrt(3.0)
    s0_ref[...] = s3 * x * y
    s1_ref[...] = s3 * y * zc
    s2_ref[...] = 0.5 * (3.0 * zc * zc - 1.0)
    s3_ref[...] = s3 * zc * x
    s4_ref[...] = 0.5 * s3 * (x * x - y * y)


def _edge_proj_body(er_ref, c_ref, w_ref, mn_ref, bt_ref, out_ref, *,
                    use_c):
    ps = er_ref.shape[1]
    z5 = jnp.zeros((5, 128), jnp.float32)
    one = jnp.ones((1, 128), jnp.float32)
    mn = mn_ref[...]
    bt = bt_ref[...]
    dn = (((0,), (0,)), ((), ()))
    for g in range(ps):
        er_r = er_ref[0, g][None, :]
        c_r = c_ref[0, g][None, :] if use_c else one
        d = er_r - mn
        rbf = jnp.exp(-bt * d * d)
        ag = jnp.concatenate([rbf * c_r if use_c else rbf, c_r, z5], axis=0)
        out_ref[pl.ds(g * 128, 128), :] = lax.dot_general(
            ag, w_ref[...], dn, preferred_element_type=jnp.float32)


def _node_feat_body(z_ref, wnbr_ref, bnbr_ref, wna_ref, bna_ref,
                    nbr_ref, na_ref):
    z = z_ref[...]
    nbr_ref[...] = jax.nn.silu(
        jnp.dot(z, wnbr_ref[...], preferred_element_type=jnp.float32)
        + bnbr_ref[...])
    na_ref[...] = jax.nn.silu(
        jnp.dot(z, wna_ref[...], preferred_element_type=jnp.float32)
        + bna_ref[...])


def _h_body(na_ref, m2_ref, wa_ref, wm_ref, b_ref, g_ref, lb_ref,
            wru_ref, bru_ref, h_ref):
    m = m2_ref[0] + m2_ref[1]
    h = (jnp.dot(na_ref[...], wa_ref[...], preferred_element_type=jnp.float32)
         + jnp.dot(m, wm_ref[...], preferred_element_type=jnp.float32)
         + b_ref[...])
    mu = jnp.mean(h, axis=-1, keepdims=True)
    var = jnp.mean((h - mu) ** 2, axis=-1, keepdims=True)
    h = (h - mu) * lax.rsqrt(var + 1e-5) * g_ref[...] + lb_ref[...]
    h = jax.nn.silu(h)
    h_ref[...] = (
        jnp.dot(h, wru_ref[...], preferred_element_type=jnp.float32)
        + bru_ref[...])


def _sc_msg_body(nbr_hbm, gate_hbm, isrc_hbm, jsrc_hbm, zeros_hbm, out_hbm,
                 jall, iv0, iv1, r0, r1, g0, g1, s0, s1, msh,
                 sg0, sg1, siv0, siv1, ssc0, ssc1, *, n_pad, dne, epw, k, nch):
    cid = lax.axis_index("c")
    sid = lax.axis_index("s")
    wid = sid * NC + cid
    rpt = n_pad // NS
    base = wid * epw
    bufs = ((iv0, r0, g0, s0, sg0, siv0, ssc0),
            (iv1, r1, g1, s1, sg1, siv1, ssc1))

    def fire(b, g):
        iv, rows, gv, sv, sg, siv, _ = bufs[b]
        off = base + g * k
        pltpu.async_copy(nbr_hbm.at[jall.at[pl.ds(g * k, k)]], rows, sg)
        pltpu.async_copy(gate_hbm.at[pl.ds(off, k)], gv, sg)

    def process(b, g, first):
        iv, rows, gv, sv, sg, siv, ssc = bufs[b]
        off = base + g * k
        pltpu.make_async_copy(
            nbr_hbm.at[jall.at[pl.ds(g * k, k)]], rows, sg).wait()
        pltpu.make_async_copy(gate_hbm.at[pl.ds(off, k)], gv, sg).wait()
        if not first:
            pltpu.make_async_copy(sv, msh.at[iv], ssc).wait()
        pltpu.async_copy(isrc_hbm.at[pl.ds(off, k)], iv, siv)

        @plsc.parallel_loop(0, k, unroll=4)
        def rowfn(rr):
            for cc in range(dne // L):
                sl = pl.ds(cc * L, L)
                sv[rr, sl] = rows[rr, sl] * gv[rr, sl]
        pltpu.make_async_copy(isrc_hbm.at[pl.ds(off, k)], iv, siv).wait()
        pltpu.async_copy(sv, msh.at[iv], ssc, add=True)

    def waitsc(b):
        iv, rows, gv, sv, sg, siv, ssc = bufs[b]
        pltpu.make_async_copy(sv, msh.at[iv], ssc).wait()

    pltpu.sync_copy(jsrc_hbm.at[pl.ds(base, epw)], jall)
    pltpu.sync_copy(zeros_hbm.at[pl.ds(sid * rpt, rpt)],
                    msh.at[pl.ds(sid * rpt, rpt)])
    fire(0, 0)
    fire(1, 1)
    plsc.subcore_barrier()
    process(0, 0, True)
    fire(0, 2)
    process(1, 1, True)
    fire(1, 3)

    def pair(p, carry):
        process(0, 2 * p, False)
        fire(0, 2 * p + 2)
        process(1, 2 * p + 1, False)
        fire(1, 2 * p + 3)
        return carry

    lax.fori_loop(1, (nch - 2) // 2, pair, 0)
    process(0, nch - 2, False)
    process(1, nch - 1, False)
    waitsc(0)
    waitsc(1)
    plsc.subcore_barrier()
    pltpu.sync_copy(msh.at[pl.ds(sid * rpt, rpt)],
                    out_hbm.at[cid, pl.ds(sid * rpt, rpt)])


def _sc_t_body(h_hbm, erp_hbm, isrc_hbm, jsrc_hbm, t_hbm,
               iall, jall, a0, a1, b0, b1, e0, e1, s0, s1,
               sg0, sg1, st0, st1, *, dne, epw, k, nch):
    cid = lax.axis_index("c")
    sid = lax.axis_index("s")
    wid = sid * NC + cid
    base = wid * epw
    bufs = ((a0, b0, e0, s0, sg0, st0), (a1, b1, e1, s1, sg1, st1))

    def fire(b, g):
        av, bv, ev, sv, sg, _ = bufs[b]
        off = base + g * k
        pltpu.async_copy(h_hbm.at[iall.at[pl.ds(g * k, k)]], av, sg)
        pltpu.async_copy(h_hbm.at[jall.at[pl.ds(g * k, k)]], bv, sg)
        pltpu.async_copy(erp_hbm.at[pl.ds(off, k)], ev, sg)

    def process(b, g, first):
        av, bv, ev, sv, sg, st = bufs[b]
        off = base + g * k
        pltpu.make_async_copy(
            h_hbm.at[iall.at[pl.ds(g * k, k)]], av, sg).wait()
        pltpu.make_async_copy(
            h_hbm.at[jall.at[pl.ds(g * k, k)]], bv, sg).wait()
        pltpu.make_async_copy(erp_hbm.at[pl.ds(off, k)], ev, sg).wait()
        if not first:
            pltpu.make_async_copy(sv, t_hbm.at[pl.ds(off, k)], st).wait()

        @plsc.parallel_loop(0, k, unroll=4)
        def rowfn(rr):
            for cc in range(dne // L):
                sl = pl.ds(cc * L, L)
                sv[rr, sl] = (av[rr, sl] + bv[rr, sl]) * ev[rr, sl]
        pltpu.async_copy(sv, t_hbm.at[pl.ds(off, k)], st)

    def waitst(b):
        av, bv, ev, sv, sg, st = bufs[b]
        pltpu.make_async_copy(sv, t_hbm.at[pl.ds(base, k)], st).wait()

    pltpu.sync_copy(isrc_hbm.at[pl.ds(base, epw)], iall)
    pltpu.sync_copy(jsrc_hbm.at[pl.ds(base, epw)], jall)
    fire(0, 0)
    fire(1, 1)
    process(0, 0, True)
    fire(0, 2)
    process(1, 1, True)
    fire(1, 3)

    def pair(p, carry):
        process(0, 2 * p, False)
        fire(0, 2 * p + 2)
        process(1, 2 * p + 1, False)
        fire(1, 2 * p + 3)
        return carry

    lax.fori_loop(1, (nch - 3) // 2, pair, 0)
    process(0, nch - 3, False)
    fire(0, nch - 1)
    process(1, nch - 2, False)
    process(0, nch - 1, False)
    waitst(1)
    waitst(0)


def kernel(z, edge, r, u, A_na_w, A_na_b, A_nbr_w, A_nbr_b, W_ndp_w, W_ndp_b,
           W_nrd_w, W_nrd_b, W_nru_w, W_nru_b, W_erp_w, W_erp_b, ln_g, ln_b,
           means, betas):
    n_nodes, zf = z.shape
    n_edges = r.shape[0]
    dne = A_na_w.shape[0]
    nrbf = means.shape[0]

    i_idx = edge[:, 0]
    j_idx = edge[:, 1]
    r2 = r[:, None]

    full = lambda s: pl.BlockSpec(s, lambda i: (0,) * len(s))

    BE = 2560
    n_eb = n_edges // BE
    PS = BE // 128
    pk = lambda a: a.reshape(n_eb, PS, 128)
    pspec = full((n_eb, PS, 128))
    er_p, c_p, s0, s1, s2, s3c, s4 = pl.pallas_call(
        _edge_scalar_body,
        grid=(1,),
        in_specs=[pspec] * 4,
        out_specs=[pspec] * 7,
        out_shape=[jax.ShapeDtypeStruct((n_eb, PS, 128), jnp.float32)] * 7,
    )(pk(r), pk(u[:, 0]), pk(u[:, 1]), pk(u[:, 2]))
    c = c_p.reshape(n_edges, 1)
    rt2 = jnp.stack([s0.reshape(-1), s1.reshape(-1), s2.reshape(-1),
                     s3c.reshape(-1), s4.reshape(-1)], axis=-1)

    def _proj(wmat, bias, use_c):
        return pl.pallas_call(
            functools.partial(_edge_proj_body, use_c=use_c),
            grid=(n_eb,),
            in_specs=[
                pl.BlockSpec((1, PS, 128), lambda i: (i, 0, 0)),
                pl.BlockSpec((1, PS, 128), lambda i: (i, 0, 0)),
                full((nrbf + 6, dne)),
                full((nrbf, 1)), full((nrbf, 1)),
            ],
            out_specs=[pl.BlockSpec((BE, dne), lambda i: (i, 0))],
            out_shape=[jax.ShapeDtypeStruct((n_edges, dne), jnp.float32)],
        )(er_p, c_p,
          jnp.concatenate([wmat.T, bias[None, :],
                           jnp.zeros((5, dne), jnp.float32)], axis=0),
          means[:, None], betas[:, None])

    (gate,) = _proj(W_ndp_w, W_ndp_b, True)
    (erp,) = _proj(W_erp_w, W_erp_b, False)

    BN = 2000
    n_nb = n_nodes // BN
    nbr, na = pl.pallas_call(
        _node_feat_body,
        grid=(n_nb,),
        in_specs=[
            pl.BlockSpec((BN, zf), lambda i: (i, 0)),
            full((zf, dne)), full((1, dne)),
            full((zf, dne)), full((1, dne)),
        ],
        out_specs=[
            pl.BlockSpec((BN, dne), lambda i: (i, 0)),
            pl.BlockSpec((BN, dne), lambda i: (i, 0)),
        ],
        out_shape=[
            jax.ShapeDtypeStruct((n_nodes, dne), jnp.float32),
            jax.ShapeDtypeStruct((n_nodes, dne), jnp.float32),
        ],
    )(z, A_nbr_w.T, A_nbr_b[None, :], A_na_w.T, A_na_b[None, :])

    EPW = n_edges // (NC * NS)
    K1 = 40
    NCH1 = EPW // K1
    K2 = 80
    NCH2 = EPW // K2
    n_pad = ((n_nodes + 8 * NS - 1) // (8 * NS)) * (8 * NS)
    mesh = plsc.VectorSubcoreMesh(core_axis_name="c", subcore_axis_name="s",
                                  num_cores=NC, num_subcores=NS)
    zeros = jnp.zeros((n_pad, dne), jnp.float32)
    m2 = pl.kernel(
        functools.partial(_sc_msg_body, n_pad=n_pad, dne=dne, epw=EPW,
                          k=K1, nch=NCH1),
        out_type=jax.ShapeDtypeStruct((NC, n_pad, dne), jnp.float32),
        mesh=mesh,
        scratch_types=[
            pltpu.VMEM((EPW,), jnp.int32),
            pltpu.VMEM((K1,), jnp.int32),
            pltpu.VMEM((K1,), jnp.int32),
            pltpu.VMEM((K1, dne), jnp.float32),
            pltpu.VMEM((K1, dne), jnp.float32),
            pltpu.VMEM((K1, dne), jnp.float32),
            pltpu.VMEM((K1, dne), jnp.float32),
            pltpu.VMEM((K1, dne), jnp.float32),
            pltpu.VMEM((K1, dne), jnp.float32),
            pltpu.VMEM_SHARED((n_pad, dne), jnp.float32),
            pltpu.SemaphoreType.DMA,
            pltpu.SemaphoreType.DMA,
            pltpu.SemaphoreType.DMA,
            pltpu.SemaphoreType.DMA,
            pltpu.SemaphoreType.DMA,
            pltpu.SemaphoreType.DMA,
        ],
    )(nbr, gate, i_idx, j_idx, zeros)

    wa = W_nrd_w[:, :dne].T
    wm = W_nrd_w[:, dne:].T
    (h,) = pl.pallas_call(
        _h_body,
        grid=(n_nb,),
        in_specs=[
            pl.BlockSpec((BN, dne), lambda i: (i, 0)),
            pl.BlockSpec((NC, BN, dne), lambda i: (0, i, 0)),
            full((dne, dne)), full((dne, dne)), full((1, dne)),
            full((1, dne)), full((1, dne)),
            full((dne, dne)), full((1, dne)),
        ],
        out_specs=[pl.BlockSpec((BN, dne), lambda i: (i, 0))],
        out_shape=[jax.ShapeDtypeStruct((n_nodes, dne), jnp.float32)],
    )(na, m2, wa, wm, W_nrd_b[None, :], ln_g[None, :], ln_b[None, :],
      W_nru_w.T, W_nru_b[None, :])

    t = pl.kernel(
        functools.partial(_sc_t_body, dne=dne, epw=EPW, k=K2, nch=NCH2),
        out_type=jax.ShapeDtypeStruct((n_edges, dne), jnp.float32),
        mesh=mesh,
        scratch_types=[
            pltpu.VMEM((EPW,), jnp.int32),
            pltpu.VMEM((EPW,), jnp.int32),
            pltpu.VMEM((K2, dne), jnp.float32),
            pltpu.VMEM((K2, dne), jnp.float32),
            pltpu.VMEM((K2, dne), jnp.float32),
            pltpu.VMEM((K2, dne), jnp.float32),
            pltpu.VMEM((K2, dne), jnp.float32),
            pltpu.VMEM((K2, dne), jnp.float32),
            pltpu.VMEM((K2, dne), jnp.float32),
            pltpu.VMEM((K2, dne), jnp.float32),
            pltpu.SemaphoreType.DMA,
            pltpu.SemaphoreType.DMA,
            pltpu.SemaphoreType.DMA,
            pltpu.SemaphoreType.DMA,
        ],
    )(h, erp, i_idx, j_idx)

    return h, t, r2, u, rt2, c
